# Initial kernel scaffold; baseline (speedup 1.0000x reference)
import jax
import jax.numpy as jnp
from jax.experimental import pallas as pl


def kernel(logits, temperatures, top_k, top_p, min_p):
    raise NotImplementedError("write your pallas kernel here")


# speedup vs baseline: 90.1907x; 90.1907x over previous
import functools

import jax
import jax.numpy as jnp
from jax.experimental import pallas as pl
from jax.experimental.pallas import tpu as pltpu

_RB = 8


def _body(logits_ref, e_ref, temp_ref, topk_ref, topp_ref, minp_ref,
          out_ref, q_ref, *, v_real):
    rb, vp = logits_ref.shape
    lane = jax.lax.broadcasted_iota(jnp.int32, (rb, vp), 1)

    l = logits_ref[...]
    temp0 = temp_ref[...]
    temp = jnp.maximum(temp0, 1e-10)

    gmax = jnp.max(l, axis=1, keepdims=True)
    garg = jnp.min(jnp.where(l == gmax, lane, vp), axis=1, keepdims=True)

    s = l / temp
    smax = jnp.max(s, axis=1, keepdims=True)
    q = jnp.exp(s - smax)
    q_ref[...] = q

    z = jnp.sum(q, axis=1, keepdims=True)
    qmax = jnp.max(q, axis=1, keepdims=True)
    minp = minp_ref[...]
    cmin = jnp.sum(jnp.where(q >= minp, 1.0, 0.0), axis=1, keepdims=True)

    tk = topk_ref[...]
    valid_k = (tk > 0) & (tk < v_real)
    n2 = jnp.where(valid_k, jnp.minimum(tk.astype(jnp.float32), cmin), cmin)
    p_mass = topp_ref[...] * z

    top_bits = jax.lax.bitcast_convert_type(qmax, jnp.int32) + 1
    lc0 = jax.lax.bitcast_convert_type(jnp.minimum(minp, qmax), jnp.int32)
    hc0 = top_bits
    ls0 = jnp.zeros_like(lc0)
    hs0 = top_bits

    def step(_, carry):
        lc, hc, ls, hs = carry
        qv = q_ref[...]
        mc = jax.lax.shift_right_logical(lc + hc, 1)
        tc = jax.lax.bitcast_convert_type(mc, jnp.float32)
        cnt = jnp.sum(jnp.where(qv >= tc, 1.0, 0.0), axis=1, keepdims=True)
        ok_c = cnt >= n2
        lc = jnp.where(ok_c, mc, lc)
        hc = jnp.where(ok_c, hc, mc)
        ms = jax.lax.shift_right_logical(ls + hs, 1)
        ts = jax.lax.bitcast_convert_type(ms, jnp.float32)
        ssum = jnp.sum(jnp.where(qv >= ts, qv, 0.0), axis=1, keepdims=True)
        ok_s = ssum <= p_mass
        hs = jnp.where(ok_s, ms, hs)
        ls = jnp.where(ok_s, ls, ms)
        return lc, hc, ls, hs

    lc, _, _, hs = jax.lax.fori_loop(0, 30, step, (lc0, hc0, ls0, hs0))
    t2 = jax.lax.bitcast_convert_type(lc, jnp.float32)
    tp = jax.lax.bitcast_convert_type(hs, jnp.float32)
    thr = jnp.maximum(t2, tp)

    qv = q_ref[...]
    keep = (qv >= thr) | (qv >= qmax)
    score = jnp.where(keep, qv / (e_ref[...] + 1e-10), -1.0)
    smax2 = jnp.max(score, axis=1, keepdims=True)
    win = jnp.min(jnp.where(score == smax2, lane, vp), axis=1, keepdims=True)

    out_ref[...] = jnp.where(temp0 == 0.0, garg, win)


def kernel(logits, temperatures, top_k, top_p, min_p):
    b, v = logits.shape
    logits = logits.astype(jnp.float32)
    e = jax.random.exponential(jax.random.key(1234), (b, v), dtype=jnp.float32)
    vp = ((v + 127) // 128) * 128
    if vp != v:
        logits = jnp.pad(logits, ((0, 0), (0, vp - v)),
                         constant_values=-jnp.inf)
        e = jnp.pad(e, ((0, 0), (0, vp - v)), constant_values=1.0)

    col = lambda x: x.reshape(b, 1)
    out = pl.pallas_call(
        functools.partial(_body, v_real=v),
        grid=(b // _RB,),
        in_specs=[
            pl.BlockSpec((_RB, vp), lambda i: (i, 0)),
            pl.BlockSpec((_RB, vp), lambda i: (i, 0)),
            pl.BlockSpec((_RB, 1), lambda i: (i, 0)),
            pl.BlockSpec((_RB, 1), lambda i: (i, 0)),
            pl.BlockSpec((_RB, 1), lambda i: (i, 0)),
            pl.BlockSpec((_RB, 1), lambda i: (i, 0)),
        ],
        out_specs=pl.BlockSpec((_RB, 1), lambda i: (i, 0)),
        out_shape=jax.ShapeDtypeStruct((b, 1), jnp.int32),
        scratch_shapes=[pltpu.VMEM((_RB, vp), jnp.float32)],
    )(logits, e, col(temperatures), col(top_k.astype(jnp.int32)),
      col(top_p), col(min_p))
    return out[:, 0]


# speedup vs baseline: 90.2024x; 1.0001x over previous
import functools

import jax
import jax.numpy as jnp
from jax.experimental import pallas as pl
from jax.experimental.pallas import tpu as pltpu

_RB = 8


def _body(logits_ref, e_ref, temp_ref, topk_ref, topp_ref, minp_ref,
          out_ref, q_ref, *, v_real):
    rb, vp = logits_ref.shape
    lane = jax.lax.broadcasted_iota(jnp.int32, (rb, vp), 1)

    l = logits_ref[...]
    temp0 = temp_ref[...]
    temp = jnp.maximum(temp0, 1e-10)

    gmax = jnp.max(l, axis=1, keepdims=True)
    garg = jnp.min(jnp.where(l == gmax, lane, vp), axis=1, keepdims=True)

    s = l / temp
    smax = jnp.max(s, axis=1, keepdims=True)
    q = jnp.exp(s - smax)
    q_ref[...] = q

    z = jnp.sum(q, axis=1, keepdims=True)
    qmax = jnp.max(q, axis=1, keepdims=True)
    minp = minp_ref[...]
    cmin = jnp.sum(jnp.where(q >= minp, 1.0, 0.0), axis=1, keepdims=True)

    tk = topk_ref[...]
    valid_k = (tk > 0) & (tk < v_real)
    n2 = jnp.where(valid_k, jnp.minimum(tk.astype(jnp.float32), cmin), cmin)
    p_mass = topp_ref[...] * z

    top_bits = jax.lax.bitcast_convert_type(qmax, jnp.int32) + 1
    lc0 = jax.lax.bitcast_convert_type(jnp.minimum(minp, qmax), jnp.int32)
    hc0 = top_bits
    ls0 = jnp.zeros_like(lc0)
    hs0 = top_bits

    def step(_, carry):
        lc, hc, ls, hs = carry
        qv = q_ref[...]
        mc = jax.lax.shift_right_logical(lc + hc, 1)
        tc = jax.lax.bitcast_convert_type(mc, jnp.float32)
        cnt = jnp.sum(jnp.where(qv >= tc, 1.0, 0.0), axis=1, keepdims=True)
        ok_c = cnt >= n2
        lc = jnp.where(ok_c, mc, lc)
        hc = jnp.where(ok_c, hc, mc)
        ms = jax.lax.shift_right_logical(ls + hs, 1)
        ts = jax.lax.bitcast_convert_type(ms, jnp.float32)
        ssum = jnp.sum(jnp.where(qv >= ts, qv, 0.0), axis=1, keepdims=True)
        ok_s = ssum <= p_mass
        hs = jnp.where(ok_s, ms, hs)
        ls = jnp.where(ok_s, ls, ms)
        return lc, hc, ls, hs

    lc, _, _, hs = jax.lax.fori_loop(0, 30, step, (lc0, hc0, ls0, hs0))
    t2 = jax.lax.bitcast_convert_type(lc, jnp.float32)
    tp = jax.lax.bitcast_convert_type(hs, jnp.float32)
    thr = jnp.maximum(t2, tp)

    qv = q_ref[...]
    keep = (qv >= thr) | (qv >= qmax)
    score = jnp.where(keep, qv / (e_ref[...] + 1e-10), -1.0)
    smax2 = jnp.max(score, axis=1, keepdims=True)
    win = jnp.min(jnp.where(score == smax2, lane, vp), axis=1, keepdims=True)

    out_ref[...] = jnp.where(temp0 == 0.0, garg, win)


def kernel(logits, temperatures, top_k, top_p, min_p):
    b, v = logits.shape
    logits = logits.astype(jnp.float32)
    e = jax.random.exponential(jax.random.key(1234), (b, v), dtype=jnp.float32)
    vp = ((v + 127) // 128) * 128
    if vp != v:
        logits = jnp.pad(logits, ((0, 0), (0, vp - v)),
                         constant_values=-jnp.inf)
        e = jnp.pad(e, ((0, 0), (0, vp - v)), constant_values=1.0)

    col = lambda x: x.reshape(b, 1)
    out = pl.pallas_call(
        functools.partial(_body, v_real=v),
        grid=(b // _RB,),
        in_specs=[
            pl.BlockSpec((_RB, vp), lambda i: (i, 0)),
            pl.BlockSpec((_RB, vp), lambda i: (i, 0)),
            pl.BlockSpec((_RB, 1), lambda i: (i, 0)),
            pl.BlockSpec((_RB, 1), lambda i: (i, 0)),
            pl.BlockSpec((_RB, 1), lambda i: (i, 0)),
            pl.BlockSpec((_RB, 1), lambda i: (i, 0)),
        ],
        out_specs=pl.BlockSpec((_RB, 1), lambda i: (i, 0)),
        out_shape=jax.ShapeDtypeStruct((b, 1), jnp.int32),
        scratch_shapes=[pltpu.VMEM((_RB, vp), jnp.float32)],
        compiler_params=pltpu.CompilerParams(
            dimension_semantics=("parallel",)),
    )(logits, e, col(temperatures), col(top_k.astype(jnp.int32)),
      col(top_p), col(min_p))
    return out[:, 0]
